# baseline (speedup 1.0000x reference)
import functools

import jax
import jax.numpy as jnp
from jax import lax
from jax.experimental import pallas as pl
from jax.experimental.pallas import tpu as pltpu
from jax.experimental.pallas import tpu_sc as plsc

N = 50000
E = 800000
HID = 64
NH = 4
CH = 16
NLAYER = 3
RW = 24

NTILE = 16
EPT = E // NTILE
KC = 400
SUB = 80
NSUB = KC // SUB
NCHUNK = EPT // KC
ROWCH = 3200
ZCH = 400


def _emb_body(x_ref, w_ref, b_ref, o_ref):
    x = x_ref[0]
    w = w_ref[0]
    o_ref[0] = jax.nn.relu(x * w + b_ref[pl.program_id(0)])


def _embed(x2, w_emb, b_emb, bn):
    grid = (2, N // bn)
    return pl.pallas_call(
        _emb_body,
        grid=grid,
        in_specs=[
            pl.BlockSpec((1, bn, 1), lambda t, i: (t, i, 0)),
            pl.BlockSpec((1, 1, HID), lambda t, i: (t, 0, 0)),
            pl.BlockSpec((2, HID), lambda t, i: (0, 0)),
        ],
        out_specs=pl.BlockSpec((1, bn, HID), lambda t, i: (t, i, 0)),
        out_shape=jax.ShapeDtypeStruct((2, N, HID), jnp.float32),
    )(x2, w_emb, b_emb)


def _prep_body(nblk, xs_ref, xd_ref, w_ref, as_ref, ad_ref,
               hs_ref, ac_ref, m_ref, mx_ref):
    i = pl.program_id(1)
    xs = xs_ref[0]
    xd = xd_ref[0]
    w = w_ref[0]
    hs = jnp.dot(xs, w, preferred_element_type=jnp.float32)
    hd = jnp.dot(xd, w, preferred_element_type=jnp.float32)
    bn = hs.shape[0]
    a_s = as_ref[0].reshape(1, HID)
    a_d = ad_ref[0].reshape(1, HID)
    als = (hs * a_s).reshape(bn, NH, CH).sum(-1)
    ald = (hd * a_d).reshape(bn, NH, CH).sum(-1)
    hs_ref[0] = hs
    ac_ref[0] = jnp.concatenate([als, ald], axis=1)
    for q in range(NH):
        ms = jnp.max(als[:, q])
        md = jnp.max(ald[:, q])

        @pl.when(i == 0)
        def _(q=q, ms=ms, md=md):
            mx_ref[q] = ms
            mx_ref[NH + q] = md

        @pl.when(i > 0)
        def _(q=q, ms=ms, md=md):
            mx_ref[q] = jnp.maximum(mx_ref[q], ms)
            mx_ref[NH + q] = jnp.maximum(mx_ref[NH + q], md)

    @pl.when(i == nblk - 1)
    def _():
        m_ref[0] = jnp.concatenate(
            [jnp.full((1, CH), jnp.maximum(mx_ref[q] + mx_ref[NH + q], 0.0))
             for q in range(NH)], axis=0)


def _prep(xs2, w3, asrc, adst, bn):
    nblk = N // bn
    grid = (3, nblk)
    return pl.pallas_call(
        functools.partial(_prep_body, nblk),
        grid=grid,
        in_specs=[
            pl.BlockSpec((1, bn, HID), lambda t, i: (t // 2, i, 0)),
            pl.BlockSpec((1, bn, HID), lambda t, i: ((t + 1) // 2, i, 0)),
            pl.BlockSpec((1, HID, HID), lambda t, i: (t, 0, 0)),
            pl.BlockSpec((1, NH, CH), lambda t, i: (t, 0, 0)),
            pl.BlockSpec((1, NH, CH), lambda t, i: (t, 0, 0)),
        ],
        out_specs=[
            pl.BlockSpec((1, bn, HID), lambda t, i: (t, i, 0)),
            pl.BlockSpec((1, bn, 8), lambda t, i: (t, i, 0)),
            pl.BlockSpec((1, NH, CH), lambda t, i: (t, 0, 0)),
        ],
        out_shape=[
            jax.ShapeDtypeStruct((3, N, HID), jnp.float32),
            jax.ShapeDtypeStruct((3, N, 8), jnp.float32),
            jax.ShapeDtypeStruct((3, NH, CH), jnp.float32),
        ],
        scratch_shapes=[pltpu.SMEM((8,), jnp.float32)],
    )(xs2, xs2, w3, asrc, adst)


def _sc_body(hs_ref, ac_ref, m_ref, src_ref, dst_ref, acc_o, ACC,
             *scr):
    srcv = scr[0:4]
    dstv = scr[4:8]
    giv = scr[8:12]
    dgv = scr[12:16]
    adr = scr[16:18]
    R = scr[18:20]
    Mb = scr[20]
    semS = scr[21:25]
    semG = scr[25:27]
    semW = scr[27:29]
    R0 = R[0]
    c = lax.axis_index("c")
    s = lax.axis_index("s")
    iota = lax.iota(jnp.int32, 16)
    zero16 = jnp.zeros((16,), jnp.float32)
    r0 = s * ROWCH
    nfull = (N - (NTILE - 1) * ROWCH) // ZCH

    pltpu.sync_copy(m_ref.at[pl.ds(2 * c, 2)], Mb)

    def _zero_r():
        def _zr(k, _):
            R0[k, pl.ds(0, 16)] = zero16
            R0[k, pl.ds(8, 16)] = zero16
            return 0
        lax.fori_loop(0, ZCH, _zr, 0)

    _zero_r()

    def _initrows(i):
        rr = r0 + i * ZCH
        pltpu.sync_copy(R0, ACC.at[pl.ds(rr, ZCH)])

    for i in range(ROWCH // ZCH):
        if i < nfull:
            _initrows(i)
        else:
            @pl.when(s < NTILE - 1)
            def _(i=i):
                _initrows(i)
    plsc.subcore_barrier()

    col16 = jnp.full((16,), CH, jnp.int32)
    col17 = jnp.full((16,), CH + 1, jnp.int32)

    for p in range(2):
        ghN = (2 * c + p) * N
        Mv = Mb[p]
        colp = jnp.full((16,), p, jnp.int32)
        ebase = s * EPT

        def _stage(j, bi):
            off = ebase + j * KC
            for q in range(NSUB):
                pltpu.async_copy(src_ref.at[pl.ds(off + q * SUB, SUB)],
                                 srcv[bi].at[q], semS[bi])
                pltpu.async_copy(dst_ref.at[pl.ds(off + q * SUB, SUB)],
                                 dstv[bi].at[q], semS[bi])

        def _wait_stage(bi):
            for q in range(NSUB):
                pltpu.make_async_copy(src_ref.at[pl.ds(0, SUB)],
                                      srcv[bi].at[q], semS[bi]).wait()
                pltpu.make_async_copy(dst_ref.at[pl.ds(0, SUB)],
                                      dstv[bi].at[q], semS[bi]).wait()

        def _build(bi, ghN=ghN):
            for q in range(NSUB):
                for l in range(SUB // 16):
                    sl = pl.ds(l * 16, 16)
                    giv[bi][q, sl] = srcv[bi][q, sl] + ghN
                    dgv[bi][q, sl] = dstv[bi][q, sl] + c * N

        def _gathers(bi, b2):
            for q in range(NSUB):
                pltpu.async_copy(hs_ref.at[giv[bi].at[q]],
                                 R[b2].at[pl.ds(q * SUB, SUB)], semG[b2])
                pltpu.async_copy(ac_ref.at[dgv[bi].at[q]],
                                 adr[b2].at[pl.ds(q * SUB, SUB)], semG[b2])

        def _wait_gathers(bi, b2):
            for q in range(NSUB):
                pltpu.make_async_copy(hs_ref.at[giv[bi].at[q]],
                                      R[b2].at[pl.ds(q * SUB, SUB)],
                                      semG[b2]).wait()
                pltpu.make_async_copy(ac_ref.at[dgv[bi].at[q]],
                                      adr[b2].at[pl.ds(q * SUB, SUB)],
                                      semG[b2]).wait()

        def _compute(b2, Mv=Mv, colp=colp):
            def _k(k, _):
                rows = k * 16 + iota
                a_s = plsc.load_gather(R[b2], [rows, col17])
                a_d = plsc.load_gather(adr[b2], [rows, colp])
                z = a_s + a_d
                e = jnp.where(z >= 0.0, z, 0.2 * z)
                w = jnp.exp(e - Mv)
                for cc in range(CH + 1):
                    cv = jnp.full((16,), cc, jnp.int32)
                    v = plsc.load_gather(R[b2], [rows, cv])
                    plsc.store_scatter(R[b2], [rows, cv], v * w)
                return 0
            lax.fori_loop(0, KC // 16, _k, 0)

        def _scatter(bi, b2):
            for q in range(NSUB):
                pltpu.async_copy(R[b2].at[pl.ds(q * SUB, SUB)],
                                 ACC.at[dstv[bi].at[q]], semW[b2], add=True)

        def _wait_scatter(bi, b2):
            for q in range(NSUB):
                pltpu.make_async_copy(R[b2].at[pl.ds(q * SUB, SUB)],
                                      ACC.at[dstv[bi].at[q]],
                                      semW[b2]).wait()

        _stage(0, 0)
        _stage(1, 1)
        _wait_stage(0)
        _build(0)
        _gathers(0, 0)

        def _group(j2, _):
            for b in range(4):
                j = 4 * j2 + b
                bi, b2 = b, b % 2
                nbi, nb2 = (b + 1) % 4, (b + 1) % 2
                _wait_gathers(bi, b2)
                _compute(b2)
                _scatter(bi, b2)
                _wait_stage(nbi)
                _build(nbi)
                if b == 0:
                    @pl.when(j2 >= 1)
                    def _():
                        _wait_scatter((b + 3) % 4, nb2)
                else:
                    _wait_scatter((b + 3) % 4, nb2)
                _gathers(nbi, nb2)
                if b == 3:
                    @pl.when(j2 < NCHUNK // 4 - 1)
                    def _(j=j):
                        _stage(j + 2, (b + 2) % 4)
                else:
                    _stage(j + 2, (b + 2) % 4)
            return 0

        lax.fori_loop(0, NCHUNK // 4, _group, 0)
        _wait_gathers(0, 0)
        _compute(0)
        _scatter(0, 0)
        _wait_scatter(3, 1)
        _wait_scatter(0, 0)
        plsc.subcore_barrier()

        def _outrows(i, p=p):
            rr = r0 + i * ZCH
            pltpu.sync_copy(ACC.at[pl.ds(rr, ZCH)], R0)
            pltpu.sync_copy(R0, acc_o.at[2 * c + p, pl.ds(rr, ZCH)])

        def _rezero(i):
            rr = r0 + i * ZCH
            pltpu.sync_copy(R0, ACC.at[pl.ds(rr, ZCH)])

        for i in range(ROWCH // ZCH):
            if i < nfull:
                _outrows(i)
            else:
                @pl.when(s < NTILE - 1)
                def _(i=i):
                    _outrows(i)
        if p == 0:
            _zero_r()
            for i in range(ROWCH // ZCH):
                if i < nfull:
                    _rezero(i)
                else:
                    @pl.when(s < NTILE - 1)
                    def _(i=i):
                        _rezero(i)
            plsc.subcore_barrier()


@functools.partial(
    pl.kernel,
    out_type=jax.ShapeDtypeStruct((NH, N, RW), jnp.float32),
    mesh=plsc.VectorSubcoreMesh(core_axis_name="c", subcore_axis_name="s"),
    compiler_params=pltpu.CompilerParams(use_tc_tiling_on_sc=False,
                                         needs_layout_passes=False),
    scratch_types=[
        pltpu.VMEM_SHARED((N, RW), jnp.float32),
    ] + [pltpu.VMEM((NSUB, SUB), jnp.int32)] * 16
    + [pltpu.VMEM((KC, 8), jnp.float32)] * 2
    + [pltpu.VMEM((KC, RW), jnp.float32)] * 2
    + [pltpu.VMEM((2, CH), jnp.float32)]
    + [pltpu.SemaphoreType.DMA] * 8,
)
def _sc_conv(hs_tab, acomb, mtab, src, dst, acc_o, *scratch):
    _sc_body(hs_tab, acomb, mtab, src, dst, acc_o, *scratch)


def _combine_body(ag_ref, a1_ref, a2_ref, b_ref, o_ref):
    def term(a_ref):
        parts = []
        for h in range(NH):
            num = a_ref[h, :, 0:CH]
            den = a_ref[h, :, CH:CH + 1] + 1e-16
            parts.append(num / den)
        return jnp.concatenate(parts, axis=1)

    b = b_ref[...]
    o_ref[0] = jax.nn.relu(term(ag_ref) + b[0])
    o_ref[1] = jax.nn.relu(term(a1_ref) + b[1] + term(a2_ref) + b[2])


def _combine(accs, bias, bn):
    grid = (N // bn,)
    a_spec = pl.BlockSpec((NH, bn, RW), lambda i: (0, i, 0))
    return pl.pallas_call(
        _combine_body,
        grid=grid,
        in_specs=[a_spec, a_spec, a_spec,
                  pl.BlockSpec((3, HID), lambda i: (0, 0))],
        out_specs=pl.BlockSpec((2, bn, HID), lambda i: (0, i, 0)),
        out_shape=jax.ShapeDtypeStruct((2, N, HID), jnp.float32),
    )(accs[0], accs[1], accs[2], bias)


def _final_body(x_ref, w_ref, b_ref, o_ref):
    o_ref[0] = (jnp.dot(x_ref[0], w_ref[0],
                        preferred_element_type=jnp.float32)
                + b_ref[pl.program_id(0)])


def _final(xs2, w_out, b_out, bn):
    grid = (2, N // bn)
    return pl.pallas_call(
        _final_body,
        grid=grid,
        in_specs=[
            pl.BlockSpec((1, bn, HID), lambda t, i: (t, i, 0)),
            pl.BlockSpec((1, HID, HID), lambda t, i: (t, 0, 0)),
            pl.BlockSpec((2, HID), lambda t, i: (0, 0)),
        ],
        out_specs=pl.BlockSpec((1, bn, HID), lambda t, i: (t, i, 0)),
        out_shape=jax.ShapeDtypeStruct((2, N, HID), jnp.float32),
    )(xs2, w_out, b_out)


def kernel(x_gene, x_protein, edge_index_gene_gene, edge_index_gene_protein,
           edge_index_protein_protein, W_emb, b_emb, W_gat, att_src, att_dst,
           b_gat, W_out, b_out):
    bn = 10000
    x2 = jnp.stack([x_gene, x_protein])
    xs = _embed(x2, W_emb, b_emb, 2000)
    eis = (edge_index_gene_gene, edge_index_gene_protein,
           edge_index_protein_protein)
    one = jnp.ones((3, NH, N, 1), jnp.float32)
    pad6 = jnp.zeros((3, NH, N, RW - CH - 2), jnp.float32)
    padc = jnp.zeros((3, 2, N, 6), jnp.float32)
    for l in range(NLAYER):
        hs3, al8, m_t = _prep(xs, W_gat[l], att_src[l], att_dst[l], 5000)
        hsh = hs3.reshape(3, N, NH, CH).transpose(0, 2, 1, 3)
        alsh = al8[:, :, :NH].transpose(0, 2, 1)[..., None]
        hs_t = jnp.concatenate([hsh, one, alsh, pad6],
                               axis=-1).reshape(3, NH * N, RW)
        aldh = al8[:, :, NH:].reshape(3, N, 2, 2).transpose(0, 2, 1, 3)
        ac_t = jnp.concatenate([aldh, padc], axis=-1).reshape(3, 2 * N, 8)
        accs = []
        for t in range(3):
            accs.append(_sc_conv(hs_t[t], ac_t[t], m_t[t],
                                 eis[t][0], eis[t][1]))
        xs = _combine(accs, b_gat[l], 1000)
    out = _final(xs, W_out, b_out, bn)
    return (out[0], out[1])


# speedup vs baseline: 1.0010x; 1.0010x over previous
import functools

import jax
import jax.numpy as jnp
from jax import lax
from jax.experimental import pallas as pl
from jax.experimental.pallas import tpu as pltpu
from jax.experimental.pallas import tpu_sc as plsc

N = 50000
E = 800000
HID = 64
NH = 4
CH = 16
NLAYER = 3
RW = 24

NTILE = 16
EPT = E // NTILE
KC = 400
SUB = 80
NSUB = KC // SUB
NCHUNK = EPT // KC
ROWCH = 3200
ZCH = 400


def _emb_body(x_ref, w_ref, b_ref, o_ref):
    x = x_ref[0]
    w = w_ref[0]
    o_ref[0] = jax.nn.relu(x * w + b_ref[pl.program_id(0)])


def _embed(x2, w_emb, b_emb, bn):
    grid = (2, N // bn)
    return pl.pallas_call(
        _emb_body,
        grid=grid,
        in_specs=[
            pl.BlockSpec((1, bn, 1), lambda t, i: (t, i, 0)),
            pl.BlockSpec((1, 1, HID), lambda t, i: (t, 0, 0)),
            pl.BlockSpec((2, HID), lambda t, i: (0, 0)),
        ],
        out_specs=pl.BlockSpec((1, bn, HID), lambda t, i: (t, i, 0)),
        out_shape=jax.ShapeDtypeStruct((2, N, HID), jnp.float32),
    )(x2, w_emb, b_emb)


def _prep_body(nblk, xs_ref, xd_ref, w_ref, as_ref, ad_ref,
               hs_ref, ac_ref, m_ref, mx_ref):
    i = pl.program_id(1)
    xs = xs_ref[0]
    xd = xd_ref[0]
    w = w_ref[0]
    hs = jnp.dot(xs, w, preferred_element_type=jnp.float32)
    hd = jnp.dot(xd, w, preferred_element_type=jnp.float32)
    bn = hs.shape[0]
    a_s = as_ref[0].reshape(1, HID)
    a_d = ad_ref[0].reshape(1, HID)
    als = (hs * a_s).reshape(bn, NH, CH).sum(-1)
    ald = (hd * a_d).reshape(bn, NH, CH).sum(-1)
    hs_ref[0] = hs
    ac_ref[0] = jnp.concatenate([als, ald], axis=1)
    for q in range(NH):
        ms = jnp.max(als[:, q])
        md = jnp.max(ald[:, q])

        @pl.when(i == 0)
        def _(q=q, ms=ms, md=md):
            mx_ref[q] = ms
            mx_ref[NH + q] = md

        @pl.when(i > 0)
        def _(q=q, ms=ms, md=md):
            mx_ref[q] = jnp.maximum(mx_ref[q], ms)
            mx_ref[NH + q] = jnp.maximum(mx_ref[NH + q], md)

    @pl.when(i == nblk - 1)
    def _():
        m_ref[0] = jnp.concatenate(
            [jnp.full((1, CH), jnp.maximum(mx_ref[q] + mx_ref[NH + q], 0.0))
             for q in range(NH)], axis=0)


def _prep(xs2, w3, asrc, adst, bn):
    nblk = N // bn
    grid = (3, nblk)
    return pl.pallas_call(
        functools.partial(_prep_body, nblk),
        grid=grid,
        in_specs=[
            pl.BlockSpec((1, bn, HID), lambda t, i: (t // 2, i, 0)),
            pl.BlockSpec((1, bn, HID), lambda t, i: ((t + 1) // 2, i, 0)),
            pl.BlockSpec((1, HID, HID), lambda t, i: (t, 0, 0)),
            pl.BlockSpec((1, NH, CH), lambda t, i: (t, 0, 0)),
            pl.BlockSpec((1, NH, CH), lambda t, i: (t, 0, 0)),
        ],
        out_specs=[
            pl.BlockSpec((1, bn, HID), lambda t, i: (t, i, 0)),
            pl.BlockSpec((1, bn, 8), lambda t, i: (t, i, 0)),
            pl.BlockSpec((1, NH, CH), lambda t, i: (t, 0, 0)),
        ],
        out_shape=[
            jax.ShapeDtypeStruct((3, N, HID), jnp.float32),
            jax.ShapeDtypeStruct((3, N, 8), jnp.float32),
            jax.ShapeDtypeStruct((3, NH, CH), jnp.float32),
        ],
        scratch_shapes=[pltpu.SMEM((8,), jnp.float32)],
    )(xs2, xs2, w3, asrc, adst)


def _sc_body(hs_ref, ac_ref, m_ref, src_ref, dst_ref, acc_o, ACC,
             *scr):
    srcv = scr[0:4]
    dstv = scr[4:8]
    giv = scr[8:12]
    dgv = scr[12:16]
    adr = scr[16:18]
    R = scr[18:20]
    Mb = scr[20]
    semS = scr[21:25]
    semG = scr[25:27]
    semW = scr[27:29]
    R0 = R[0]
    c = lax.axis_index("c")
    s = lax.axis_index("s")
    iota = lax.iota(jnp.int32, 16)
    zero16 = jnp.zeros((16,), jnp.float32)
    r0 = s * ROWCH
    nfull = (N - (NTILE - 1) * ROWCH) // ZCH

    pltpu.sync_copy(m_ref.at[pl.ds(2 * c, 2)], Mb)

    def _zero_r():
        def _zr(k, _):
            R0[k, pl.ds(0, 16)] = zero16
            R0[k, pl.ds(8, 16)] = zero16
            return 0
        lax.fori_loop(0, ZCH, _zr, 0)

    _zero_r()

    def _initrows(i):
        rr = r0 + i * ZCH
        pltpu.sync_copy(R0, ACC.at[pl.ds(rr, ZCH)])

    for i in range(ROWCH // ZCH):
        if i < nfull:
            _initrows(i)
        else:
            @pl.when(s < NTILE - 1)
            def _(i=i):
                _initrows(i)
    plsc.subcore_barrier()

    col16 = jnp.full((16,), CH, jnp.int32)
    col17 = jnp.full((16,), CH + 1, jnp.int32)

    for p in range(2):
        ghN = (2 * c + p) * N
        Mv = Mb[p]
        colp = jnp.full((16,), p, jnp.int32)
        ebase = s * EPT

        def _stage(j, bi):
            off = ebase + j * KC
            for q in range(NSUB):
                pltpu.async_copy(src_ref.at[pl.ds(off + q * SUB, SUB)],
                                 srcv[bi].at[q], semS[bi])
                pltpu.async_copy(dst_ref.at[pl.ds(off + q * SUB, SUB)],
                                 dstv[bi].at[q], semS[bi])

        def _wait_stage(bi):
            for q in range(NSUB):
                pltpu.make_async_copy(src_ref.at[pl.ds(0, SUB)],
                                      srcv[bi].at[q], semS[bi]).wait()
                pltpu.make_async_copy(dst_ref.at[pl.ds(0, SUB)],
                                      dstv[bi].at[q], semS[bi]).wait()

        def _build(bi, ghN=ghN):
            for q in range(NSUB):
                for l in range(SUB // 16):
                    sl = pl.ds(l * 16, 16)
                    giv[bi][q, sl] = srcv[bi][q, sl] + ghN
                    dgv[bi][q, sl] = dstv[bi][q, sl] + c * N

        def _gathers(bi, b2):
            for q in range(NSUB):
                pltpu.async_copy(hs_ref.at[giv[bi].at[q]],
                                 R[b2].at[pl.ds(q * SUB, SUB)], semG[b2])
                pltpu.async_copy(ac_ref.at[dgv[bi].at[q]],
                                 adr[b2].at[pl.ds(q * SUB, SUB)], semG[b2])

        def _wait_gathers(bi, b2):
            for q in range(NSUB):
                pltpu.make_async_copy(hs_ref.at[giv[bi].at[q]],
                                      R[b2].at[pl.ds(q * SUB, SUB)],
                                      semG[b2]).wait()
                pltpu.make_async_copy(ac_ref.at[dgv[bi].at[q]],
                                      adr[b2].at[pl.ds(q * SUB, SUB)],
                                      semG[b2]).wait()

        def _compute(b2, Mv=Mv, colp=colp):
            def _k(k, _):
                rows = k * 16 + iota
                a_s = plsc.load_gather(R[b2], [rows, col17])
                a_d = plsc.load_gather(adr[b2], [rows, colp])
                z = a_s + a_d
                e = jnp.where(z >= 0.0, z, 0.2 * z)
                w = jnp.exp(e - Mv)
                for cc in range(CH + 1):
                    cv = jnp.full((16,), cc, jnp.int32)
                    v = plsc.load_gather(R[b2], [rows, cv])
                    plsc.store_scatter(R[b2], [rows, cv], v * w)
                return 0
            lax.fori_loop(0, KC // 16, _k, 0)

        def _scatter(bi, b2):
            for q in range(NSUB):
                pltpu.async_copy(R[b2].at[pl.ds(q * SUB, SUB)],
                                 ACC.at[dstv[bi].at[q]], semW[b2], add=True)

        def _wait_scatter(bi, b2):
            for q in range(NSUB):
                pltpu.make_async_copy(R[b2].at[pl.ds(q * SUB, SUB)],
                                      ACC.at[dstv[bi].at[q]],
                                      semW[b2]).wait()

        _stage(0, 0)
        _stage(1, 1)
        _wait_stage(0)
        _build(0)
        _gathers(0, 0)

        def _group(j2, _):
            for b in range(4):
                j = 4 * j2 + b
                bi, b2 = b, b % 2
                nbi, nb2 = (b + 1) % 4, (b + 1) % 2
                _wait_gathers(bi, b2)
                _compute(b2)
                _scatter(bi, b2)
                _wait_stage(nbi)
                _build(nbi)
                if b == 0:
                    @pl.when(j2 >= 1)
                    def _():
                        _wait_scatter((b + 3) % 4, nb2)
                else:
                    _wait_scatter((b + 3) % 4, nb2)
                _gathers(nbi, nb2)
                if b == 3:
                    @pl.when(j2 < NCHUNK // 4 - 1)
                    def _(j=j):
                        _stage(j + 2, (b + 2) % 4)
                else:
                    _stage(j + 2, (b + 2) % 4)
            return 0

        lax.fori_loop(0, NCHUNK // 4, _group, 0)
        _wait_gathers(0, 0)
        _compute(0)
        _scatter(0, 0)
        _wait_scatter(3, 1)
        _wait_scatter(0, 0)
        plsc.subcore_barrier()

        def _outrows(i, p=p):
            rr = r0 + i * ZCH
            pltpu.sync_copy(ACC.at[pl.ds(rr, ZCH)],
                            acc_o.at[2 * c + p, pl.ds(rr, ZCH)])

        def _rezero(i):
            rr = r0 + i * ZCH
            pltpu.sync_copy(R0, ACC.at[pl.ds(rr, ZCH)])

        for i in range(ROWCH // ZCH):
            if i < nfull:
                _outrows(i)
            else:
                @pl.when(s < NTILE - 1)
                def _(i=i):
                    _outrows(i)
        if p == 0:
            _zero_r()
            for i in range(ROWCH // ZCH):
                if i < nfull:
                    _rezero(i)
                else:
                    @pl.when(s < NTILE - 1)
                    def _(i=i):
                        _rezero(i)
            plsc.subcore_barrier()


@functools.partial(
    pl.kernel,
    out_type=jax.ShapeDtypeStruct((NH, N, RW), jnp.float32),
    mesh=plsc.VectorSubcoreMesh(core_axis_name="c", subcore_axis_name="s"),
    compiler_params=pltpu.CompilerParams(use_tc_tiling_on_sc=False,
                                         needs_layout_passes=False),
    scratch_types=[
        pltpu.VMEM_SHARED((N, RW), jnp.float32),
    ] + [pltpu.VMEM((NSUB, SUB), jnp.int32)] * 16
    + [pltpu.VMEM((KC, 8), jnp.float32)] * 2
    + [pltpu.VMEM((KC, RW), jnp.float32)] * 2
    + [pltpu.VMEM((2, CH), jnp.float32)]
    + [pltpu.SemaphoreType.DMA] * 8,
)
def _sc_conv(hs_tab, acomb, mtab, src, dst, acc_o, *scratch):
    _sc_body(hs_tab, acomb, mtab, src, dst, acc_o, *scratch)


def _combine_body(ag_ref, a1_ref, a2_ref, b_ref, o_ref):
    def term(a_ref):
        parts = []
        for h in range(NH):
            num = a_ref[h, :, 0:CH]
            den = a_ref[h, :, CH:CH + 1] + 1e-16
            parts.append(num / den)
        return jnp.concatenate(parts, axis=1)

    b = b_ref[...]
    o_ref[0] = jax.nn.relu(term(ag_ref) + b[0])
    o_ref[1] = jax.nn.relu(term(a1_ref) + b[1] + term(a2_ref) + b[2])


def _combine(accs, bias, bn):
    grid = (N // bn,)
    a_spec = pl.BlockSpec((NH, bn, RW), lambda i: (0, i, 0))
    return pl.pallas_call(
        _combine_body,
        grid=grid,
        in_specs=[a_spec, a_spec, a_spec,
                  pl.BlockSpec((3, HID), lambda i: (0, 0))],
        out_specs=pl.BlockSpec((2, bn, HID), lambda i: (0, i, 0)),
        out_shape=jax.ShapeDtypeStruct((2, N, HID), jnp.float32),
    )(accs[0], accs[1], accs[2], bias)


def _final_body(x_ref, w_ref, b_ref, o_ref):
    o_ref[0] = (jnp.dot(x_ref[0], w_ref[0],
                        preferred_element_type=jnp.float32)
                + b_ref[pl.program_id(0)])


def _final(xs2, w_out, b_out, bn):
    grid = (2, N // bn)
    return pl.pallas_call(
        _final_body,
        grid=grid,
        in_specs=[
            pl.BlockSpec((1, bn, HID), lambda t, i: (t, i, 0)),
            pl.BlockSpec((1, HID, HID), lambda t, i: (t, 0, 0)),
            pl.BlockSpec((2, HID), lambda t, i: (0, 0)),
        ],
        out_specs=pl.BlockSpec((1, bn, HID), lambda t, i: (t, i, 0)),
        out_shape=jax.ShapeDtypeStruct((2, N, HID), jnp.float32),
    )(xs2, w_out, b_out)


def kernel(x_gene, x_protein, edge_index_gene_gene, edge_index_gene_protein,
           edge_index_protein_protein, W_emb, b_emb, W_gat, att_src, att_dst,
           b_gat, W_out, b_out):
    bn = 10000
    x2 = jnp.stack([x_gene, x_protein])
    xs = _embed(x2, W_emb, b_emb, 2000)
    eis = (edge_index_gene_gene, edge_index_gene_protein,
           edge_index_protein_protein)
    one = jnp.ones((3, NH, N, 1), jnp.float32)
    pad6 = jnp.zeros((3, NH, N, RW - CH - 2), jnp.float32)
    padc = jnp.zeros((3, 2, N, 6), jnp.float32)
    for l in range(NLAYER):
        hs3, al8, m_t = _prep(xs, W_gat[l], att_src[l], att_dst[l], 5000)
        hsh = hs3.reshape(3, N, NH, CH).transpose(0, 2, 1, 3)
        alsh = al8[:, :, :NH].transpose(0, 2, 1)[..., None]
        hs_t = jnp.concatenate([hsh, one, alsh, pad6],
                               axis=-1).reshape(3, NH * N, RW)
        aldh = al8[:, :, NH:].reshape(3, N, 2, 2).transpose(0, 2, 1, 3)
        ac_t = jnp.concatenate([aldh, padc], axis=-1).reshape(3, 2 * N, 8)
        accs = []
        for t in range(3):
            accs.append(_sc_conv(hs_t[t], ac_t[t], m_t[t],
                                 eis[t][0], eis[t][1]))
        xs = _combine(accs, b_gat[l], 1000)
    out = _final(xs, W_out, b_out, bn)
    return (out[0], out[1])
